# baseline (speedup 1.0000x reference)
import functools

import jax
import jax.numpy as jnp
from jax import lax
from jax.experimental import pallas as pl
from jax.experimental.pallas import tpu as pltpu
from jax.experimental.pallas import tpu_sc as plsc

H = 64
W128 = 128
QUART = 12288
NQ = 6
NSUB = 16
CH = 128


def _lin_body(x_ref, w_ref, b_ref, o_ref, *, relu):
    y = jnp.dot(x_ref[...], w_ref[...], preferred_element_type=jnp.float32)
    y = y + b_ref[...]
    o_ref[...] = jnp.maximum(y, 0.0) if relu else y


def _lin(x, w, b, relu, block=512):
    R, K = x.shape
    N = w.shape[1]
    return pl.pallas_call(
        functools.partial(_lin_body, relu=relu),
        grid=(R // block,),
        in_specs=[
            pl.BlockSpec((block, K), lambda i: (i, 0)),
            pl.BlockSpec((K, N), lambda i: (0, 0)),
            pl.BlockSpec((1, N), lambda i: (0, 0)),
        ],
        out_specs=pl.BlockSpec((block, N), lambda i: (i, 0)),
        out_shape=jax.ShapeDtypeStruct((R, N), jnp.float32),
    )(x, w, b)


def _upd2_body(x1_ref, agg_ref, cnt_ref, w_ref, b_ref, bm_ref, o_ref, *, nba):
    pid = pl.program_id(0)
    c = cnt_ref[...]
    recip = 1.0 / jnp.maximum(c, 1.0)
    m = jnp.minimum(c, 1.0)
    x2 = agg_ref[...][:, :H] * recip + m * bm_ref[...]
    x2 = jnp.where(pid < nba, x2, 0.0)
    x = jnp.concatenate([x1_ref[...], x2], axis=1)
    y = jnp.dot(x, w_ref[...], preferred_element_type=jnp.float32) + b_ref[...]
    o_ref[...] = jnp.maximum(y, 0.0)


def _upd2(x1, agg, cnt, w, b, bm, block=512):
    R = x1.shape[0]
    nba = agg.shape[0] // block
    return pl.pallas_call(
        functools.partial(_upd2_body, nba=nba),
        grid=(R // block,),
        in_specs=[
            pl.BlockSpec((block, H), lambda i: (i, 0)),
            pl.BlockSpec((block, W128), lambda i: (jnp.minimum(i, nba - 1), 0)),
            pl.BlockSpec((block, 1), lambda i: (i, 0)),
            pl.BlockSpec((2 * H, H), lambda i: (0, 0)),
            pl.BlockSpec((1, H), lambda i: (0, 0)),
            pl.BlockSpec((1, H), lambda i: (0, 0)),
        ],
        out_specs=pl.BlockSpec((block, H), lambda i: (i, 0)),
        out_shape=jax.ShapeDtypeStruct((R, H), jnp.float32),
    )(x1, agg, cnt, w, b, bm)


def _lin_pad_body(x_ref, w_ref, o_ref, *, nb):
    pid = pl.program_id(0)
    y = jnp.dot(x_ref[...], w_ref[...], preferred_element_type=jnp.float32)
    o_ref[...] = jnp.where(pid < nb, y, 0.0)


def _wpad(w):
    return jnp.concatenate(
        [w, jnp.zeros((w.shape[0], W128 - w.shape[1]), jnp.float32)], axis=1)


def _lin_pad(x, w, block=512):
    R, K = x.shape
    N = w.shape[1]
    nb = R // block
    return pl.pallas_call(
        functools.partial(_lin_pad_body, nb=nb),
        grid=(nb + 1,),
        in_specs=[
            pl.BlockSpec((block, K), lambda i: (jnp.minimum(i, nb - 1), 0)),
            pl.BlockSpec((K, N), lambda i: (0, 0)),
        ],
        out_specs=pl.BlockSpec((block, N), lambda i: (i, 0)),
        out_shape=jax.ShapeDtypeStruct((R + block, N), jnp.float32),
    )(x, w)


def _sc_segsum(table, gidx, sidx, zrows):
    E = gidx.shape[0] // NQ
    epw = E // NSUB
    nch = epw // CH
    zr = zrows.shape[0]
    mesh = plsc.VectorSubcoreMesh(core_axis_name="c", subcore_axis_name="s")

    @functools.partial(
        pl.kernel, mesh=mesh,
        out_type=jax.ShapeDtypeStruct((NQ * QUART, W128), jnp.float32),
        scratch_types=[
            pltpu.VMEM((CH,), jnp.int32),
            pltpu.VMEM((CH,), jnp.int32),
            pltpu.VMEM((CH, W128), jnp.float32),
            pltpu.VMEM_SHARED((QUART, W128), jnp.float32),
            pltpu.SemaphoreType.DMA,
        ],
    )
    def k(table_h, gidx_h, sidx_h, z_h, out_h, gv, sv, rows, acc, sem):
        cid = lax.axis_index("c")
        sid = lax.axis_index("s")
        for j in range(NQ // 2):
            q = (NQ // 2) * cid + j
            pltpu.sync_copy(z_h, acc.at[pl.ds(sid * zr, zr)])
            plsc.subcore_barrier()
            base0 = q * E + sid * epw

            def body(i, carry):
                base = base0 + i * CH
                pltpu.sync_copy(gidx_h.at[pl.ds(base, CH)], gv)
                pltpu.sync_copy(sidx_h.at[pl.ds(base, CH)], sv)
                pltpu.async_copy(table_h.at[gv], rows, sem).wait()
                pltpu.sync_copy(rows, acc.at[sv], add=True)
                return carry

            lax.fori_loop(0, nch, body, 0)
            plsc.subcore_barrier()
            pltpu.sync_copy(acc.at[pl.ds(sid * zr, zr)],
                            out_h.at[pl.ds(q * QUART + sid * zr, zr)])
            plsc.subcore_barrier()

    return k(table, gidx, sidx, zrows)


def _mk_idx(src, dst, zero_row):
    gs, ss = [], []
    for c in range(NQ):
        lo = c * QUART
        inr = (dst >= lo) & (dst < lo + QUART)
        gs.append(jnp.where(inr, src, zero_row))
        ss.append(jnp.clip(dst - lo, 0, QUART - 1))
    return (jnp.concatenate(gs).astype(jnp.int32),
            jnp.concatenate(ss).astype(jnp.int32))


def _upd_body(x1_ref, x2_ref, w_ref, b_ref, o_ref):
    x = jnp.concatenate([x1_ref[...], x2_ref[...]], axis=1)
    y = jnp.dot(x, w_ref[...], preferred_element_type=jnp.float32) + b_ref[...]
    o_ref[...] = jnp.maximum(y, 0.0)


def _upd(x1, x2, w, b, block=512):
    R = x1.shape[0]
    return pl.pallas_call(
        _upd_body,
        grid=(R // block,),
        in_specs=[
            pl.BlockSpec((block, H), lambda i: (i, 0)),
            pl.BlockSpec((block, H), lambda i: (i, 0)),
            pl.BlockSpec((2 * H, H), lambda i: (0, 0)),
            pl.BlockSpec((1, H), lambda i: (0, 0)),
        ],
        out_specs=pl.BlockSpec((block, H), lambda i: (i, 0)),
        out_shape=jax.ShapeDtypeStruct((R, H), jnp.float32),
    )(x1, x2, w, b)


def _tt_body(t_ref, mn_ref, wm_ref, be_ref, o_ref, *, nb, T):
    tb = t_ref[...]
    mn = mn_ref[...]
    wm = wm_ref[...]
    outs = []
    for j in range(nb):
        a = jnp.dot(mn, tb[j], preferred_element_type=jnp.float32)
        outs.append(jnp.dot(a, wm, preferred_element_type=jnp.float32)[None])
    o_ref[...] = jnp.concatenate(outs, axis=0) + be_ref[...]


def _tt(tile3, mn, wm, beff, nb=8):
    B, T, _ = tile3.shape
    return pl.pallas_call(
        functools.partial(_tt_body, nb=nb, T=T),
        grid=(B // nb,),
        in_specs=[
            pl.BlockSpec((nb, T, H), lambda i: (i, 0, 0)),
            pl.BlockSpec((T, T), lambda i: (0, 0)),
            pl.BlockSpec((H, H), lambda i: (0, 0)),
            pl.BlockSpec((1, T, H), lambda i: (0, 0, 0)),
        ],
        out_specs=pl.BlockSpec((nb, T, H), lambda i: (i, 0, 0)),
        out_shape=jax.ShapeDtypeStruct((B, T, H), jnp.float32),
    )(tile3, mn, wm, beff)


def _global_body(t_ref, gh_ref, wg_ref, bg_ref, wu_ref, bu_ref, wm_ref,
                 bm_ref, gh_o, gm_o):
    tmean = jnp.mean(t_ref[...], axis=1)
    ga = jnp.dot(tmean, wg_ref[...], preferred_element_type=jnp.float32) + bg_ref[...]
    x = jnp.concatenate([gh_ref[...], ga], axis=1)
    ghn = jnp.maximum(
        jnp.dot(x, wu_ref[...], preferred_element_type=jnp.float32) + bu_ref[...], 0.0)
    gh_o[...] = ghn
    gm_o[...] = jnp.dot(ghn, wm_ref[...], preferred_element_type=jnp.float32) + bm_ref[...]


def _global(tile3, gh, wg, bg, wu, bu, wm, bm, nb=256):
    B, T, _ = tile3.shape
    nb = min(nb, B)
    return pl.pallas_call(
        _global_body,
        grid=(B // nb,),
        in_specs=[
            pl.BlockSpec((nb, T, H), lambda i: (i, 0, 0)),
            pl.BlockSpec((nb, H), lambda i: (i, 0)),
            pl.BlockSpec((H, H), lambda i: (0, 0)),
            pl.BlockSpec((1, H), lambda i: (0, 0)),
            pl.BlockSpec((2 * H, H), lambda i: (0, 0)),
            pl.BlockSpec((1, H), lambda i: (0, 0)),
            pl.BlockSpec((H, H), lambda i: (0, 0)),
            pl.BlockSpec((1, H), lambda i: (0, 0)),
        ],
        out_specs=[
            pl.BlockSpec((nb, H), lambda i: (i, 0)),
            pl.BlockSpec((nb, H), lambda i: (i, 0)),
        ],
        out_shape=[
            jax.ShapeDtypeStruct((B, H), jnp.float32),
            jax.ShapeDtypeStruct((B, H), jnp.float32),
        ],
    )(tile3, gh, wg, bg, wu, bu, wm, bm)


def _updg_body(t_ref, g_ref, w_ref, b_ref, o_ref, *, nb, T):
    tb = t_ref[...]
    g3 = jnp.broadcast_to(g_ref[...][:, None, :], (nb, T, H))
    x = jnp.concatenate([tb, g3], axis=2).reshape(nb * T, 2 * H)
    y = jnp.dot(x, w_ref[...], preferred_element_type=jnp.float32) + b_ref[...]
    o_ref[...] = jnp.maximum(y, 0.0).reshape(nb, T, H)


def _updg(tile3, gmsg, w, b, nb=8):
    B, T, _ = tile3.shape
    return pl.pallas_call(
        functools.partial(_updg_body, nb=nb, T=T),
        grid=(B // nb,),
        in_specs=[
            pl.BlockSpec((nb, T, H), lambda i: (i, 0, 0)),
            pl.BlockSpec((nb, H), lambda i: (i, 0)),
            pl.BlockSpec((2 * H, H), lambda i: (0, 0)),
            pl.BlockSpec((1, H), lambda i: (0, 0)),
        ],
        out_specs=pl.BlockSpec((nb, T, H), lambda i: (i, 0, 0)),
        out_shape=jax.ShapeDtypeStruct((B, T, H), jnp.float32),
    )(tile3, gmsg, w, b)


def _readout_body(t_ref, p_ref, pi_ref, gh_ref, gf_ref, w0_ref, b0_ref,
                  w1_ref, b1_ref, w2_ref, b2_ref, o_ref):
    tp = jnp.mean(t_ref[...], axis=1)
    pp = (jnp.mean(p_ref[...], axis=1) + jnp.mean(pi_ref[...], axis=1)) * 0.5
    comb = jnp.concatenate([tp, pp, gh_ref[...], gf_ref[...]], axis=1)
    h = jnp.maximum(
        jnp.dot(comb, w0_ref[...], preferred_element_type=jnp.float32) + b0_ref[...], 0.0)
    h = jnp.maximum(
        jnp.dot(h, w1_ref[...], preferred_element_type=jnp.float32) + b1_ref[...], 0.0)
    o_ref[...] = jnp.dot(h, w2_ref[...], preferred_element_type=jnp.float32) + b2_ref[...]


def _readout(tile3, piece3, piece_init3, gh, gf, r0, r1, r2, nb=256):
    B, T, _ = tile3.shape
    nb = min(nb, B)
    P = piece3.shape[1]
    GF = gf.shape[1]
    D0 = 3 * H + GF
    return pl.pallas_call(
        _readout_body,
        grid=(B // nb,),
        in_specs=[
            pl.BlockSpec((nb, T, H), lambda i: (i, 0, 0)),
            pl.BlockSpec((nb, P, H), lambda i: (i, 0, 0)),
            pl.BlockSpec((nb, P, H), lambda i: (i, 0, 0)),
            pl.BlockSpec((nb, H), lambda i: (i, 0)),
            pl.BlockSpec((nb, GF), lambda i: (i, 0)),
            pl.BlockSpec((D0, H), lambda i: (0, 0)),
            pl.BlockSpec((1, H), lambda i: (0, 0)),
            pl.BlockSpec((H, 32), lambda i: (0, 0)),
            pl.BlockSpec((1, 32), lambda i: (0, 0)),
            pl.BlockSpec((32, 1), lambda i: (0, 0)),
            pl.BlockSpec((1, 1), lambda i: (0, 0)),
        ],
        out_specs=pl.BlockSpec((nb, 1), lambda i: (i, 0)),
        out_shape=jax.ShapeDtypeStruct((B, 1), jnp.float32),
    )(tile3, piece3, piece_init3, gh, gf, r0["w"], r0["b"][None],
      r1["w"], r1["b"][None], r2["w"], r2["b"][None])


def kernel(tile_feats, piece_feats, global_feats, tile_edge_index,
           piece_to_tile, tile_to_piece, B, T, P, params):
    del B, T, P
    Bs, Ts, TF = tile_feats.shape
    Ps = piece_feats.shape[1]
    BT = Bs * Ts
    BP = Bs * Ps
    E = piece_to_tile.shape[1]

    t2p_src, t2p_dst = tile_to_piece[0], tile_to_piece[1]
    p2t_src, p2t_dst = piece_to_tile[0], piece_to_tile[1]

    g_t2p, s_t2p = _mk_idx(t2p_src, t2p_dst, BP)
    g_p2t, s_p2t = _mk_idx(p2t_src, p2t_dst, BP)
    zrows = jnp.zeros((QUART // NSUB, W128), jnp.float32)

    ones_table = jnp.concatenate(
        [jnp.ones((BP, W128), jnp.float32),
         jnp.zeros((512, W128), jnp.float32)], axis=0)
    cnt_p = _sc_segsum(ones_table, g_t2p, s_t2p, zrows)[:, :1]
    cnt_t_half = _sc_segsum(ones_table, g_p2t, s_p2t, zrows)[:, :1]
    cnt_t = jnp.concatenate(
        [cnt_t_half,
         jnp.zeros((BT - cnt_t_half.shape[0], 1), jnp.float32)], axis=0)

    src_tt, dst_tt = tile_edge_index[0], tile_edge_index[1]
    ar = jnp.arange(Ts, dtype=jnp.int32)
    ohs = (src_tt[:, None] == ar[None, :]).astype(jnp.float32)
    ohd = (dst_tt[:, None] == ar[None, :]).astype(jnp.float32)
    M = ohd.T @ ohs
    cnt_tt = M.sum(axis=1)
    mn = M / jnp.maximum(cnt_tt, 1.0)[:, None]
    bscale_tt = jnp.minimum(cnt_tt, 1.0)

    te, pe = params["tile_embed"], params["piece_embed"]
    tile_flat = _lin(tile_feats.reshape(BT, TF), te["w"], te["b"][None], relu=True)
    piece_flat = _lin(piece_feats.reshape(BP, -1), pe["w"], pe["b"][None], relu=True)
    piece_init = piece_flat
    gh = jnp.broadcast_to(params["global_embed"], (Bs, H))

    for p in params["mp"]:
        w_tp, b_tp = p["tile_to_piece_msg"]["w"], p["tile_to_piece_msg"]["b"]
        proj = _lin_pad(tile_flat[:BP], _wpad(w_tp))
        agg_p = _sc_segsum(proj, g_t2p, s_t2p, zrows)
        pu = p["piece_update"]
        piece_flat = _upd2(piece_flat, agg_p, cnt_p, pu["w"], pu["b"][None], b_tp[None])

        w_pt, b_pt = p["piece_to_tile_msg"]["w"], p["piece_to_tile_msg"]["b"]
        proj2 = _lin_pad(piece_flat, _wpad(w_pt))
        agg_t = _sc_segsum(proj2, g_p2t, s_p2t, zrows)
        tu = p["tile_update_pieces"]
        tile_flat = _upd2(tile_flat, agg_t, cnt_t, tu["w"], tu["b"][None], b_pt[None])

        tile3 = tile_flat.reshape(Bs, Ts, H)
        wm_tt, bm_tt = p["tile_to_tile_msg"]["w"], p["tile_to_tile_msg"]["b"]
        beff = (bscale_tt[:, None] * bm_tt[None, :])[None]
        x2tt = _tt(tile3, mn, wm_tt, beff)
        tt_u = p["tile_update_tiles"]
        tile_flat = _upd(tile_flat, x2tt.reshape(BT, H), tt_u["w"], tt_u["b"][None])
        tile3 = tile_flat.reshape(Bs, Ts, H)

        gmsg_p = p["tile_to_global_msg"]
        gu = p["global_update"]
        g2t = p["global_to_tile_msg"]
        gh, gmsg = _global(tile3, gh, gmsg_p["w"], gmsg_p["b"][None],
                           gu["w"], gu["b"][None], g2t["w"], g2t["b"][None])
        tg_u = p["tile_update_global"]
        tile3 = _updg(tile3, gmsg, tg_u["w"], tg_u["b"][None])
        tile_flat = tile3.reshape(BT, H)

    r0, r1, r2 = params["readout"]
    value = _readout(tile_flat.reshape(Bs, Ts, H), piece_flat.reshape(Bs, Ps, H),
                     piece_init.reshape(Bs, Ps, H), gh, global_feats, r0, r1, r2)
    return value[:, 0]


# speedup vs baseline: 1.0001x; 1.0001x over previous
import functools

import jax
import jax.numpy as jnp
from jax import lax
from jax.experimental import pallas as pl
from jax.experimental.pallas import tpu as pltpu
from jax.experimental.pallas import tpu_sc as plsc

H = 64
W128 = 128
QUART = 12288
NQ = 6
NSUB = 16
CH = 128


def _lin_body(x_ref, w_ref, b_ref, o_ref, *, relu):
    y = jnp.dot(x_ref[...], w_ref[...], preferred_element_type=jnp.float32)
    y = y + b_ref[...]
    o_ref[...] = jnp.maximum(y, 0.0) if relu else y


def _lin(x, w, b, relu, block=512):
    R, K = x.shape
    N = w.shape[1]
    return pl.pallas_call(
        functools.partial(_lin_body, relu=relu),
        grid=(R // block,),
        in_specs=[
            pl.BlockSpec((block, K), lambda i: (i, 0)),
            pl.BlockSpec((K, N), lambda i: (0, 0)),
            pl.BlockSpec((1, N), lambda i: (0, 0)),
        ],
        out_specs=pl.BlockSpec((block, N), lambda i: (i, 0)),
        out_shape=jax.ShapeDtypeStruct((R, N), jnp.float32),
    )(x, w, b)


def _upd2_body(x1_ref, agg_ref, cnt_ref, w_ref, b_ref, bm_ref, o_ref, *, nba):
    pid = pl.program_id(0)
    c = cnt_ref[...]
    recip = 1.0 / jnp.maximum(c, 1.0)
    m = jnp.minimum(c, 1.0)
    x2 = agg_ref[...][:, :H] * recip + m * bm_ref[...]
    x2 = jnp.where(pid < nba, x2, 0.0)
    x = jnp.concatenate([x1_ref[...], x2], axis=1)
    y = jnp.dot(x, w_ref[...], preferred_element_type=jnp.float32) + b_ref[...]
    o_ref[...] = jnp.maximum(y, 0.0)


def _upd2(x1, agg, cnt, w, b, bm, block=512):
    R = x1.shape[0]
    nba = agg.shape[0] // block
    return pl.pallas_call(
        functools.partial(_upd2_body, nba=nba),
        grid=(R // block,),
        in_specs=[
            pl.BlockSpec((block, H), lambda i: (i, 0)),
            pl.BlockSpec((block, W128), lambda i: (jnp.minimum(i, nba - 1), 0)),
            pl.BlockSpec((block, 1), lambda i: (i, 0)),
            pl.BlockSpec((2 * H, H), lambda i: (0, 0)),
            pl.BlockSpec((1, H), lambda i: (0, 0)),
            pl.BlockSpec((1, H), lambda i: (0, 0)),
        ],
        out_specs=pl.BlockSpec((block, H), lambda i: (i, 0)),
        out_shape=jax.ShapeDtypeStruct((R, H), jnp.float32),
    )(x1, agg, cnt, w, b, bm)


def _lin_pad_body(x_ref, w_ref, o_ref, *, nb):
    pid = pl.program_id(0)
    y = jnp.dot(x_ref[...], w_ref[...], preferred_element_type=jnp.float32)
    o_ref[...] = jnp.where(pid < nb, y, 0.0)


def _wpad(w):
    return jnp.concatenate(
        [w, jnp.zeros((w.shape[0], W128 - w.shape[1]), jnp.float32)], axis=1)


def _lin_pad(x, w, block=512):
    R, K = x.shape
    N = w.shape[1]
    nb = R // block
    return pl.pallas_call(
        functools.partial(_lin_pad_body, nb=nb),
        grid=(nb + 1,),
        in_specs=[
            pl.BlockSpec((block, K), lambda i: (jnp.minimum(i, nb - 1), 0)),
            pl.BlockSpec((K, N), lambda i: (0, 0)),
        ],
        out_specs=pl.BlockSpec((block, N), lambda i: (i, 0)),
        out_shape=jax.ShapeDtypeStruct((R + block, N), jnp.float32),
    )(x, w)


def _sc_segsum(table, gidx, sidx, zrows):
    E = gidx.shape[0] // NQ
    epw = E // NSUB
    nch = epw // CH
    zr = zrows.shape[0]
    mesh = plsc.VectorSubcoreMesh(core_axis_name="c", subcore_axis_name="s")
    gidx2 = gidx.reshape(-1, CH)
    sidx2 = sidx.reshape(-1, CH)

    @functools.partial(
        pl.kernel, mesh=mesh,
        out_type=jax.ShapeDtypeStruct((NQ * QUART, W128), jnp.float32),
        scratch_types=[
            pltpu.VMEM((nch, CH), jnp.int32),
            pltpu.VMEM((nch, CH), jnp.int32),
            pltpu.VMEM((CH, W128), jnp.float32),
            pltpu.VMEM_SHARED((QUART, W128), jnp.float32),
            pltpu.SemaphoreType.DMA,
        ],
    )
    def k(table_h, gidx_h, sidx_h, z_h, out_h, gv, sv, rows, acc, sem):
        cid = lax.axis_index("c")
        sid = lax.axis_index("s")
        for j in range(NQ // 2):
            q = (NQ // 2) * cid + j
            pltpu.sync_copy(z_h, acc.at[pl.ds(sid * zr, zr)])
            row0 = (q * NSUB + sid) * nch
            pltpu.sync_copy(gidx_h.at[pl.ds(row0, nch)], gv)
            pltpu.sync_copy(sidx_h.at[pl.ds(row0, nch)], sv)
            plsc.subcore_barrier()

            def body(i, carry):
                pltpu.async_copy(table_h.at[gv.at[i]], rows, sem).wait()
                pltpu.sync_copy(rows, acc.at[sv.at[i]], add=True)
                return carry

            lax.fori_loop(0, nch, body, 0)
            plsc.subcore_barrier()
            pltpu.sync_copy(acc.at[pl.ds(sid * zr, zr)],
                            out_h.at[pl.ds(q * QUART + sid * zr, zr)])
            plsc.subcore_barrier()

    return k(table, gidx2, sidx2, zrows)


def _mk_idx(src, dst, zero_row):
    gs, ss = [], []
    for c in range(NQ):
        lo = c * QUART
        inr = (dst >= lo) & (dst < lo + QUART)
        gs.append(jnp.where(inr, src, zero_row))
        ss.append(jnp.clip(dst - lo, 0, QUART - 1))
    return (jnp.concatenate(gs).astype(jnp.int32),
            jnp.concatenate(ss).astype(jnp.int32))


def _upd_body(x1_ref, x2_ref, w_ref, b_ref, o_ref):
    x = jnp.concatenate([x1_ref[...], x2_ref[...]], axis=1)
    y = jnp.dot(x, w_ref[...], preferred_element_type=jnp.float32) + b_ref[...]
    o_ref[...] = jnp.maximum(y, 0.0)


def _upd(x1, x2, w, b, block=512):
    R = x1.shape[0]
    return pl.pallas_call(
        _upd_body,
        grid=(R // block,),
        in_specs=[
            pl.BlockSpec((block, H), lambda i: (i, 0)),
            pl.BlockSpec((block, H), lambda i: (i, 0)),
            pl.BlockSpec((2 * H, H), lambda i: (0, 0)),
            pl.BlockSpec((1, H), lambda i: (0, 0)),
        ],
        out_specs=pl.BlockSpec((block, H), lambda i: (i, 0)),
        out_shape=jax.ShapeDtypeStruct((R, H), jnp.float32),
    )(x1, x2, w, b)


def _tt_body(t_ref, mn_ref, wm_ref, be_ref, o_ref, *, nb, T):
    tb = t_ref[...]
    mn = mn_ref[...]
    wm = wm_ref[...]
    outs = []
    for j in range(nb):
        a = jnp.dot(mn, tb[j], preferred_element_type=jnp.float32)
        outs.append(jnp.dot(a, wm, preferred_element_type=jnp.float32)[None])
    o_ref[...] = jnp.concatenate(outs, axis=0) + be_ref[...]


def _tt(tile3, mn, wm, beff, nb=8):
    B, T, _ = tile3.shape
    return pl.pallas_call(
        functools.partial(_tt_body, nb=nb, T=T),
        grid=(B // nb,),
        in_specs=[
            pl.BlockSpec((nb, T, H), lambda i: (i, 0, 0)),
            pl.BlockSpec((T, T), lambda i: (0, 0)),
            pl.BlockSpec((H, H), lambda i: (0, 0)),
            pl.BlockSpec((1, T, H), lambda i: (0, 0, 0)),
        ],
        out_specs=pl.BlockSpec((nb, T, H), lambda i: (i, 0, 0)),
        out_shape=jax.ShapeDtypeStruct((B, T, H), jnp.float32),
    )(tile3, mn, wm, beff)


def _global_body(t_ref, gh_ref, wg_ref, bg_ref, wu_ref, bu_ref, wm_ref,
                 bm_ref, gh_o, gm_o):
    tmean = jnp.mean(t_ref[...], axis=1)
    ga = jnp.dot(tmean, wg_ref[...], preferred_element_type=jnp.float32) + bg_ref[...]
    x = jnp.concatenate([gh_ref[...], ga], axis=1)
    ghn = jnp.maximum(
        jnp.dot(x, wu_ref[...], preferred_element_type=jnp.float32) + bu_ref[...], 0.0)
    gh_o[...] = ghn
    gm_o[...] = jnp.dot(ghn, wm_ref[...], preferred_element_type=jnp.float32) + bm_ref[...]


def _global(tile3, gh, wg, bg, wu, bu, wm, bm, nb=256):
    B, T, _ = tile3.shape
    nb = min(nb, B)
    return pl.pallas_call(
        _global_body,
        grid=(B // nb,),
        in_specs=[
            pl.BlockSpec((nb, T, H), lambda i: (i, 0, 0)),
            pl.BlockSpec((nb, H), lambda i: (i, 0)),
            pl.BlockSpec((H, H), lambda i: (0, 0)),
            pl.BlockSpec((1, H), lambda i: (0, 0)),
            pl.BlockSpec((2 * H, H), lambda i: (0, 0)),
            pl.BlockSpec((1, H), lambda i: (0, 0)),
            pl.BlockSpec((H, H), lambda i: (0, 0)),
            pl.BlockSpec((1, H), lambda i: (0, 0)),
        ],
        out_specs=[
            pl.BlockSpec((nb, H), lambda i: (i, 0)),
            pl.BlockSpec((nb, H), lambda i: (i, 0)),
        ],
        out_shape=[
            jax.ShapeDtypeStruct((B, H), jnp.float32),
            jax.ShapeDtypeStruct((B, H), jnp.float32),
        ],
    )(tile3, gh, wg, bg, wu, bu, wm, bm)


def _updg_body(t_ref, g_ref, w_ref, b_ref, o_ref, *, nb, T):
    tb = t_ref[...]
    g3 = jnp.broadcast_to(g_ref[...][:, None, :], (nb, T, H))
    x = jnp.concatenate([tb, g3], axis=2).reshape(nb * T, 2 * H)
    y = jnp.dot(x, w_ref[...], preferred_element_type=jnp.float32) + b_ref[...]
    o_ref[...] = jnp.maximum(y, 0.0).reshape(nb, T, H)


def _updg(tile3, gmsg, w, b, nb=8):
    B, T, _ = tile3.shape
    return pl.pallas_call(
        functools.partial(_updg_body, nb=nb, T=T),
        grid=(B // nb,),
        in_specs=[
            pl.BlockSpec((nb, T, H), lambda i: (i, 0, 0)),
            pl.BlockSpec((nb, H), lambda i: (i, 0)),
            pl.BlockSpec((2 * H, H), lambda i: (0, 0)),
            pl.BlockSpec((1, H), lambda i: (0, 0)),
        ],
        out_specs=pl.BlockSpec((nb, T, H), lambda i: (i, 0, 0)),
        out_shape=jax.ShapeDtypeStruct((B, T, H), jnp.float32),
    )(tile3, gmsg, w, b)


def _readout_body(t_ref, p_ref, pi_ref, gh_ref, gf_ref, w0_ref, b0_ref,
                  w1_ref, b1_ref, w2_ref, b2_ref, o_ref):
    tp = jnp.mean(t_ref[...], axis=1)
    pp = (jnp.mean(p_ref[...], axis=1) + jnp.mean(pi_ref[...], axis=1)) * 0.5
    comb = jnp.concatenate([tp, pp, gh_ref[...], gf_ref[...]], axis=1)
    h = jnp.maximum(
        jnp.dot(comb, w0_ref[...], preferred_element_type=jnp.float32) + b0_ref[...], 0.0)
    h = jnp.maximum(
        jnp.dot(h, w1_ref[...], preferred_element_type=jnp.float32) + b1_ref[...], 0.0)
    o_ref[...] = jnp.dot(h, w2_ref[...], preferred_element_type=jnp.float32) + b2_ref[...]


def _readout(tile3, piece3, piece_init3, gh, gf, r0, r1, r2, nb=256):
    B, T, _ = tile3.shape
    nb = min(nb, B)
    P = piece3.shape[1]
    GF = gf.shape[1]
    D0 = 3 * H + GF
    return pl.pallas_call(
        _readout_body,
        grid=(B // nb,),
        in_specs=[
            pl.BlockSpec((nb, T, H), lambda i: (i, 0, 0)),
            pl.BlockSpec((nb, P, H), lambda i: (i, 0, 0)),
            pl.BlockSpec((nb, P, H), lambda i: (i, 0, 0)),
            pl.BlockSpec((nb, H), lambda i: (i, 0)),
            pl.BlockSpec((nb, GF), lambda i: (i, 0)),
            pl.BlockSpec((D0, H), lambda i: (0, 0)),
            pl.BlockSpec((1, H), lambda i: (0, 0)),
            pl.BlockSpec((H, 32), lambda i: (0, 0)),
            pl.BlockSpec((1, 32), lambda i: (0, 0)),
            pl.BlockSpec((32, 1), lambda i: (0, 0)),
            pl.BlockSpec((1, 1), lambda i: (0, 0)),
        ],
        out_specs=pl.BlockSpec((nb, 1), lambda i: (i, 0)),
        out_shape=jax.ShapeDtypeStruct((B, 1), jnp.float32),
    )(tile3, piece3, piece_init3, gh, gf, r0["w"], r0["b"][None],
      r1["w"], r1["b"][None], r2["w"], r2["b"][None])


def kernel(tile_feats, piece_feats, global_feats, tile_edge_index,
           piece_to_tile, tile_to_piece, B, T, P, params):
    del B, T, P
    Bs, Ts, TF = tile_feats.shape
    Ps = piece_feats.shape[1]
    BT = Bs * Ts
    BP = Bs * Ps
    E = piece_to_tile.shape[1]

    t2p_src, t2p_dst = tile_to_piece[0], tile_to_piece[1]
    p2t_src, p2t_dst = piece_to_tile[0], piece_to_tile[1]

    g_t2p, s_t2p = _mk_idx(t2p_src, t2p_dst, BP)
    g_p2t, s_p2t = _mk_idx(p2t_src, p2t_dst, BP)
    zrows = jnp.zeros((QUART // NSUB, W128), jnp.float32)

    ones_table = jnp.concatenate(
        [jnp.ones((BP, W128), jnp.float32),
         jnp.zeros((512, W128), jnp.float32)], axis=0)
    cnt_p = _sc_segsum(ones_table, g_t2p, s_t2p, zrows)[:, :1]
    cnt_t_half = _sc_segsum(ones_table, g_p2t, s_p2t, zrows)[:, :1]
    cnt_t = jnp.concatenate(
        [cnt_t_half,
         jnp.zeros((BT - cnt_t_half.shape[0], 1), jnp.float32)], axis=0)

    src_tt, dst_tt = tile_edge_index[0], tile_edge_index[1]
    ar = jnp.arange(Ts, dtype=jnp.int32)
    ohs = (src_tt[:, None] == ar[None, :]).astype(jnp.float32)
    ohd = (dst_tt[:, None] == ar[None, :]).astype(jnp.float32)
    M = ohd.T @ ohs
    cnt_tt = M.sum(axis=1)
    mn = M / jnp.maximum(cnt_tt, 1.0)[:, None]
    bscale_tt = jnp.minimum(cnt_tt, 1.0)

    te, pe = params["tile_embed"], params["piece_embed"]
    tile_flat = _lin(tile_feats.reshape(BT, TF), te["w"], te["b"][None], relu=True)
    piece_flat = _lin(piece_feats.reshape(BP, -1), pe["w"], pe["b"][None], relu=True)
    piece_init = piece_flat
    gh = jnp.broadcast_to(params["global_embed"], (Bs, H))

    for p in params["mp"]:
        w_tp, b_tp = p["tile_to_piece_msg"]["w"], p["tile_to_piece_msg"]["b"]
        proj = _lin_pad(tile_flat[:BP], _wpad(w_tp))
        agg_p = _sc_segsum(proj, g_t2p, s_t2p, zrows)
        pu = p["piece_update"]
        piece_flat = _upd2(piece_flat, agg_p, cnt_p, pu["w"], pu["b"][None], b_tp[None])

        w_pt, b_pt = p["piece_to_tile_msg"]["w"], p["piece_to_tile_msg"]["b"]
        proj2 = _lin_pad(piece_flat, _wpad(w_pt))
        agg_t = _sc_segsum(proj2, g_p2t, s_p2t, zrows)
        tu = p["tile_update_pieces"]
        tile_flat = _upd2(tile_flat, agg_t, cnt_t, tu["w"], tu["b"][None], b_pt[None])

        tile3 = tile_flat.reshape(Bs, Ts, H)
        wm_tt, bm_tt = p["tile_to_tile_msg"]["w"], p["tile_to_tile_msg"]["b"]
        beff = (bscale_tt[:, None] * bm_tt[None, :])[None]
        x2tt = _tt(tile3, mn, wm_tt, beff)
        tt_u = p["tile_update_tiles"]
        tile_flat = _upd(tile_flat, x2tt.reshape(BT, H), tt_u["w"], tt_u["b"][None])
        tile3 = tile_flat.reshape(Bs, Ts, H)

        gmsg_p = p["tile_to_global_msg"]
        gu = p["global_update"]
        g2t = p["global_to_tile_msg"]
        gh, gmsg = _global(tile3, gh, gmsg_p["w"], gmsg_p["b"][None],
                           gu["w"], gu["b"][None], g2t["w"], g2t["b"][None])
        tg_u = p["tile_update_global"]
        tile3 = _updg(tile3, gmsg, tg_u["w"], tg_u["b"][None])
        tile_flat = tile3.reshape(BT, H)

    r0, r1, r2 = params["readout"]
    value = _readout(tile_flat.reshape(Bs, Ts, H), piece_flat.reshape(Bs, Ps, H),
                     piece_init.reshape(Bs, Ps, H), gh, global_feats, r0, r1, r2)
    return value[:, 0]
